# baseline (device time: 18495 ns/iter reference)
import jax
import jax.numpy as jnp
from jax import lax
from jax.experimental import pallas as pl
from jax.experimental.pallas import tpu as pltpu

X_SIZE = 2
WIDTHS = (128, 256, 256, 256, 256, 256, 256, 256, 128)


def kernel(x, dy):
    k_per, m = x.shape
    _, f = dy.shape
    blk = m // X_SIZE
    half = blk // 2
    assert sum(WIDTHS) == f
    K = len(WIDTHS)
    offs = [sum(WIDTHS[:i]) for i in range(K)]

    def body(x_ref, dy_ref, out_ref, xs_ref, xm_ref, send_ref, rva_ref,
             bsend_ref, brecv_ref, sem_a_s, sem_a_r, sem_b_s, sem_b_r):
        px = lax.axis_index("x")
        py = lax.axis_index("y")
        pz = lax.axis_index("z")
        h = lax.rem(pz, 2)
        partner = (1 - px, py, pz)
        znbr = (px, py, pz + 1 - 2 * h)

        barrier_sem = pltpu.get_barrier_semaphore()
        for nbr in (partner, znbr):
            pl.semaphore_signal(barrier_sem, inc=1, device_id=nbr,
                                device_id_type=pl.DeviceIdType.MESH)

        for pv in (0, 1):
            for hv in (0, 1):
                @pl.when(jnp.logical_and(px == pv, h == hv))
                def _(pv=pv, hv=hv):
                    mo = pv * blk + hv * half
                    so = (1 - pv) * blk + hv * half
                    xm_ref[:, :] = x_ref[:, mo:mo + half]
                    xs_ref[:, :] = x_ref[:, so:so + half]

        def send_gemm(c):
            sl = pl.ds(offs[c], WIDTHS[c])
            send_ref[:, sl] = lax.dot_general(
                xs_ref[:, :], dy_ref[:, sl],
                dimension_numbers=(((0,), (0,)), ((), ())),
                preferred_element_type=jnp.float32,
            ).astype(jnp.bfloat16)

        def mine_gemm(c):
            sl = pl.ds(offs[c], WIDTHS[c])
            out_ref[pl.ds(h * half, half), sl] = lax.dot_general(
                xm_ref[:, :], dy_ref[:, sl],
                dimension_numbers=(((0,), (0,)), ((), ())),
                preferred_element_type=jnp.float32,
            )

        def start_a(c):
            sl = pl.ds(offs[c], WIDTHS[c])
            a = pltpu.make_async_remote_copy(
                src_ref=send_ref.at[:, sl],
                dst_ref=rva_ref.at[:, sl],
                send_sem=sem_a_s.at[c],
                recv_sem=sem_a_r.at[c],
                device_id=partner,
                device_id_type=pl.DeviceIdType.MESH,
            )
            a.start()
            return a

        send_gemm(0)
        pl.semaphore_wait(barrier_sem, 2)

        a_copies = [start_a(0)]
        mine_gemm(0)
        for c in range(1, K):
            send_gemm(c)
            a_copies.append(start_a(c))
            mine_gemm(c)

        b_copies = []
        for c in range(K):
            sl = pl.ds(offs[c], WIDTHS[c])
            a_copies[c].wait_recv()
            summed = (out_ref[pl.ds(h * half, half), sl]
                      + rva_ref[:, sl].astype(jnp.float32))
            out_ref[pl.ds(h * half, half), sl] = summed
            bsend_ref[:, sl] = summed.astype(jnp.bfloat16)
            b = pltpu.make_async_remote_copy(
                src_ref=bsend_ref.at[:, sl],
                dst_ref=brecv_ref.at[:, sl],
                send_sem=sem_b_s.at[c],
                recv_sem=sem_b_r.at[c],
                device_id=znbr,
                device_id_type=pl.DeviceIdType.MESH,
            )
            b.start()
            b_copies.append(b)

        for c in range(K):
            sl = pl.ds(offs[c], WIDTHS[c])
            b_copies[c].wait_recv()
            out_ref[pl.ds((1 - h) * half, half), sl] = (
                brecv_ref[:, sl].astype(jnp.float32))
        for c in range(K):
            a_copies[c].wait_send()
            b_copies[c].wait_send()

    return pl.pallas_call(
        body,
        out_shape=jax.ShapeDtypeStruct((blk, f), jnp.float32),
        in_specs=[
            pl.BlockSpec(memory_space=pltpu.VMEM),
            pl.BlockSpec(memory_space=pltpu.VMEM),
        ],
        out_specs=pl.BlockSpec(memory_space=pltpu.VMEM),
        scratch_shapes=[
            pltpu.VMEM((k_per, half), jnp.float32),
            pltpu.VMEM((k_per, half), jnp.float32),
            pltpu.VMEM((half, f), jnp.bfloat16),
            pltpu.VMEM((half, f), jnp.bfloat16),
            pltpu.VMEM((half, f), jnp.bfloat16),
            pltpu.VMEM((half, f), jnp.bfloat16),
            pltpu.SemaphoreType.DMA((K,)),
            pltpu.SemaphoreType.DMA((K,)),
            pltpu.SemaphoreType.DMA((K,)),
            pltpu.SemaphoreType.DMA((K,)),
        ],
        compiler_params=pltpu.CompilerParams(collective_id=0),
    )(x, dy)


# device time: 18490 ns/iter; 1.0003x vs baseline; 1.0003x over previous
import jax
import jax.numpy as jnp
from jax import lax
from jax.experimental import pallas as pl
from jax.experimental.pallas import tpu as pltpu

X_SIZE = 2
WIDTHS = (128, 256, 256, 256, 256, 256, 256, 256, 128)


def kernel(x, dy):
    k_per, m = x.shape
    _, f = dy.shape
    blk = m // X_SIZE
    half = blk // 2
    assert sum(WIDTHS) == f
    K = len(WIDTHS)
    offs = [sum(WIDTHS[:i]) for i in range(K)]

    def body(x_ref, dy_ref, out_ref, xs_ref, xm_ref, dyb_ref, send_ref,
             rva_ref, bsend_ref, brecv_ref, sem_a_s, sem_a_r, sem_b_s,
             sem_b_r):
        px = lax.axis_index("x")
        py = lax.axis_index("y")
        pz = lax.axis_index("z")
        h = lax.rem(pz, 2)
        partner = (1 - px, py, pz)
        znbr = (px, py, pz + 1 - 2 * h)

        barrier_sem = pltpu.get_barrier_semaphore()
        for nbr in (partner, znbr):
            pl.semaphore_signal(barrier_sem, inc=1, device_id=nbr,
                                device_id_type=pl.DeviceIdType.MESH)

        for pv in (0, 1):
            for hv in (0, 1):
                @pl.when(jnp.logical_and(px == pv, h == hv))
                def _(pv=pv, hv=hv):
                    mo = pv * blk + hv * half
                    so = (1 - pv) * blk + hv * half
                    xm_ref[:, :] = x_ref[:, mo:mo + half].astype(jnp.bfloat16)
                    xs_ref[:, :] = x_ref[:, so:so + half].astype(jnp.bfloat16)
        dyb_ref[:, :] = dy_ref[:, :].astype(jnp.bfloat16)

        def send_gemm(c):
            sl = pl.ds(offs[c], WIDTHS[c])
            send_ref[:, sl] = lax.dot_general(
                xs_ref[:, :], dyb_ref[:, sl],
                dimension_numbers=(((0,), (0,)), ((), ())),
                preferred_element_type=jnp.float32,
            ).astype(jnp.bfloat16)

        def mine_gemm(c):
            sl = pl.ds(offs[c], WIDTHS[c])
            out_ref[pl.ds(h * half, half), sl] = lax.dot_general(
                xm_ref[:, :], dyb_ref[:, sl],
                dimension_numbers=(((0,), (0,)), ((), ())),
                preferred_element_type=jnp.float32,
            )

        def start_a(c):
            sl = pl.ds(offs[c], WIDTHS[c])
            a = pltpu.make_async_remote_copy(
                src_ref=send_ref.at[:, sl],
                dst_ref=rva_ref.at[:, sl],
                send_sem=sem_a_s.at[c],
                recv_sem=sem_a_r.at[c],
                device_id=partner,
                device_id_type=pl.DeviceIdType.MESH,
            )
            a.start()
            return a

        send_gemm(0)
        pl.semaphore_wait(barrier_sem, 2)

        a_copies = [start_a(0)]
        mine_gemm(0)
        for c in range(1, K):
            send_gemm(c)
            a_copies.append(start_a(c))
            mine_gemm(c)

        b_copies = []
        for c in range(K):
            sl = pl.ds(offs[c], WIDTHS[c])
            a_copies[c].wait_recv()
            summed = (out_ref[pl.ds(h * half, half), sl]
                      + rva_ref[:, sl].astype(jnp.float32))
            out_ref[pl.ds(h * half, half), sl] = summed
            bsend_ref[:, sl] = summed.astype(jnp.bfloat16)
            b = pltpu.make_async_remote_copy(
                src_ref=bsend_ref.at[:, sl],
                dst_ref=brecv_ref.at[:, sl],
                send_sem=sem_b_s.at[c],
                recv_sem=sem_b_r.at[c],
                device_id=znbr,
                device_id_type=pl.DeviceIdType.MESH,
            )
            b.start()
            b_copies.append(b)

        for c in range(K):
            sl = pl.ds(offs[c], WIDTHS[c])
            b_copies[c].wait_recv()
            out_ref[pl.ds((1 - h) * half, half), sl] = (
                brecv_ref[:, sl].astype(jnp.float32))
        for c in range(K):
            a_copies[c].wait_send()
            b_copies[c].wait_send()

    return pl.pallas_call(
        body,
        out_shape=jax.ShapeDtypeStruct((blk, f), jnp.float32),
        in_specs=[
            pl.BlockSpec(memory_space=pltpu.VMEM),
            pl.BlockSpec(memory_space=pltpu.VMEM),
        ],
        out_specs=pl.BlockSpec(memory_space=pltpu.VMEM),
        scratch_shapes=[
            pltpu.VMEM((k_per, half), jnp.bfloat16),
            pltpu.VMEM((k_per, half), jnp.bfloat16),
            pltpu.VMEM((k_per, f), jnp.bfloat16),
            pltpu.VMEM((half, f), jnp.bfloat16),
            pltpu.VMEM((half, f), jnp.bfloat16),
            pltpu.VMEM((half, f), jnp.bfloat16),
            pltpu.VMEM((half, f), jnp.bfloat16),
            pltpu.SemaphoreType.DMA((K,)),
            pltpu.SemaphoreType.DMA((K,)),
            pltpu.SemaphoreType.DMA((K,)),
            pltpu.SemaphoreType.DMA((K,)),
        ],
        compiler_params=pltpu.CompilerParams(collective_id=0),
    )(x, dy)
